# baseline (speedup 1.0000x reference)
import jax
import jax.numpy as jnp
from jax import lax
from jax.experimental import pallas as pl
from jax.experimental.pallas import tpu as pltpu
from jax.experimental.pallas import tpu_sc as plsc

NPL = 100000
NST = 50000
DF = 128
H = 64
NLAYERS = 3

NC = 2
NS = 16
CHUNK = 256
IDXW = 128
R = 25000
R_PAD = 26624
WB = 125
_MESH = plsc.VectorSubcoreMesh(core_axis_name="c", subcore_axis_name="s")


def _fill(ref, nrows, val):
    def body(i, _):
        for j in range(H // 16):
            ref[i, pl.ds(j * 16, 16)] = jnp.full((16,), val, jnp.float32)
        return 0
    lax.fori_loop(0, nrows, body, 0)


def _make_segsum(e_pad, n_dst, nq, counts, zero_upper=False):
    chunks_per_tile = e_pad // (NS * CHUNK)
    wb_per_q = R // WB
    B = 256

    def body(h, srci, dsti, out, acc, srcv, dstv, csrc, cd2, srow, d2row,
             rows, sem):
        c = lax.axis_index("c")
        s = lax.axis_index("s")
        my_rows = R_PAD // NS
        iota = lax.iota(jnp.int32, 16)

        def fire(off):
            for j2 in range(B // IDXW):
                for i in range(IDXW // 16):
                    d2row[j2, pl.ds(i * 16, 16)] = cd2[
                        pl.ds(off + j2 * IDXW + i * 16, 16)]
            if not counts:
                for j2 in range(B // IDXW):
                    for i in range(IDXW // 16):
                        srow[j2, pl.ds(i * 16, 16)] = csrc[
                            pl.ds(off + j2 * IDXW + i * 16, 16)]
                cps = [pltpu.async_copy(h.at[srow.at[j2]],
                                        rows.at[pl.ds(j2 * IDXW, IDXW)], sem)
                       for j2 in range(B // IDXW)]
                for cp in cps:
                    cp.wait()
            for j2 in range(B // IDXW):
                pltpu.sync_copy(rows.at[pl.ds(j2 * IDXW, IDXW)],
                                acc.at[d2row.at[j2]], add=True)

        for q in range(nq):
            base = (q * NC + c) * R
            _fill(rows, 128, 0.0)
            for z in range(my_rows // 128):
                pltpu.sync_copy(rows.at[pl.ds(0, 128)],
                                acc.at[pl.ds(s * my_rows + z * 128, 128)])
            if counts:
                _fill(rows, B, 1.0)
            plsc.subcore_barrier()

            def echunk(k, noff):
                rb = (s * chunks_per_tile + k) * (CHUNK // IDXW)
                if not counts:
                    pltpu.sync_copy(srci.at[pl.ds(rb, CHUNK // IDXW)], srcv)
                pltpu.sync_copy(dsti.at[pl.ds(rb, CHUNK // IDXW)], dstv)
                for j in range(CHUNK // IDXW):
                    for i in range(IDXW // 16):
                        d = dstv[j, pl.ds(i * 16, 16)] - base
                        ok = (d >= 0) & (d < R)
                        lane = lax.iota(jnp.int32, 16)
                        _, dsrt, _ = plsc.sort_key_val(lane, d, mask=ok)
                        cd2[pl.ds(noff, 16)] = dsrt
                        if not counts:
                            _, ssrt, _ = plsc.sort_key_val(
                                lane, srcv[j, pl.ds(i * 16, 16)], mask=ok)
                            csrc[pl.ds(noff, 16)] = ssrt
                        noff = noff + jnp.sum(ok.astype(jnp.int32))
                @pl.when(noff >= B)
                def _():
                    fire(0)
                @pl.when(noff >= 2 * B)
                def _():
                    fire(B)
                drained = jnp.where(noff >= 2 * B, 2 * B,
                                    jnp.where(noff >= B, B, 0))
                for i in range(B // 16):
                    v = cd2[pl.ds(drained + i * 16, 16)]
                    cd2[pl.ds(i * 16, 16)] = v
                    if not counts:
                        v2 = csrc[pl.ds(drained + i * 16, 16)]
                        csrc[pl.ds(i * 16, 16)] = v2
                return noff - drained
            noff = lax.fori_loop(0, chunks_per_tile, echunk, jnp.int32(0))
            for i in range(B // 16):
                lane = i * 16 + iota
                keep = lane < noff
                junk = R + s * 84 + i * 16 + iota
                cd2[pl.ds(i * 16, 16)] = jnp.where(keep,
                                                   cd2[pl.ds(i * 16, 16)], junk)
                if not counts:
                    csrc[pl.ds(i * 16, 16)] = jnp.where(
                        keep, csrc[pl.ds(i * 16, 16)], 0)
            fire(0)
            plsc.subcore_barrier()

            for k in range((wb_per_q + NS - 1) // NS):
                cid = k * NS + s
                @pl.when(cid < wb_per_q)
                def _():
                    if counts:
                        pltpu.sync_copy(acc.at[pl.ds(cid * WB, WB)],
                                        rows.at[pl.ds(0, WB)])
                        def inv(i, _):
                            for j in range(H // 16):
                                x = rows[i, pl.ds(j * 16, 16)]
                                rows[i, pl.ds(j * 16, 16)] = 1.0 / jnp.maximum(x, 1.0)
                            return 0
                        lax.fori_loop(0, WB, inv, 0)
                        pltpu.sync_copy(rows.at[pl.ds(0, WB)],
                                        out.at[pl.ds(base + cid * WB, WB)])
                    else:
                        pltpu.sync_copy(acc.at[pl.ds(cid * WB, WB)],
                                        out.at[pl.ds(base + cid * WB, WB)])
            plsc.subcore_barrier()

        if zero_upper:
            zlen = (n_dst - nq * NC * R) // NC
            zbase = nq * NC * R + c * zlen
            _fill(rows, WB, 0.0)
            for k in range((zlen // WB + NS - 1) // NS):
                cid = k * NS + s
                @pl.when(cid < zlen // WB)
                def _():
                    pltpu.sync_copy(rows.at[pl.ds(0, WB)],
                                    out.at[pl.ds(zbase + cid * WB, WB)])

    return pl.kernel(
        body,
        out_type=jax.ShapeDtypeStruct((n_dst, H), jnp.float32),
        mesh=_MESH,
        compiler_params=pltpu.CompilerParams(use_tc_tiling_on_sc=False,
                                             needs_layout_passes=False),
        scratch_types=[
            pltpu.VMEM_SHARED((R_PAD, H), jnp.float32),
            pltpu.VMEM((CHUNK // IDXW, IDXW), jnp.int32),
            pltpu.VMEM((CHUNK // IDXW, IDXW), jnp.int32),
            pltpu.VMEM((528,), jnp.int32),
            pltpu.VMEM((528,), jnp.int32),
            pltpu.VMEM((2, 128), jnp.int32),
            pltpu.VMEM((2, 128), jnp.int32),
            pltpu.VMEM((256, H), jnp.float32),
            pltpu.SemaphoreType.DMA,
        ],
    )


def _pad_edges(ei, e_pad):
    src = ei[0]
    dst = ei[1]
    e = src.shape[0]
    pad = e_pad - e
    src = jnp.pad(src, (0, pad))
    dst = jnp.pad(dst, (0, pad), constant_values=jnp.int32(1 << 28))
    return src.reshape(e_pad // IDXW, IDXW), dst.reshape(e_pad // IDXW, IDXW)


BM = 2000


def _proj_body(x, w, b, o):
    o[...] = jnp.maximum(jnp.dot(x[...], w[...],
                                 preferred_element_type=jnp.float32) + b[...], 0.0)


def _tc_proj(x, w, b, n):
    return pl.pallas_call(
        _proj_body,
        grid=(n // BM,),
        in_specs=[pl.BlockSpec((BM, DF), lambda i: (i, 0)),
                  pl.BlockSpec((DF, H), lambda i: (0, 0)),
                  pl.BlockSpec((1, H), lambda i: (0, 0))],
        out_specs=pl.BlockSpec((BM, H), lambda i: (i, 0)),
        out_shape=jax.ShapeDtypeStruct((n, H), jnp.float32),
    )(x, w, b)


def _sage1_body(agg, inv, h, wl, bl, wr, o):
    m = agg[...] * inv[...]
    o[...] = jnp.maximum(
        jnp.dot(m, wl[...], preferred_element_type=jnp.float32) + bl[...]
        + jnp.dot(h[...], wr[...], preferred_element_type=jnp.float32), 0.0)


def _tc_sage1(agg, inv, h, wl, bl, wr, n):
    bs = pl.BlockSpec((BM, H), lambda i: (i, 0))
    ws = pl.BlockSpec((H, H), lambda i: (0, 0))
    return pl.pallas_call(
        _sage1_body,
        grid=(n // BM,),
        in_specs=[bs, bs, bs, ws, pl.BlockSpec((1, H), lambda i: (0, 0)), ws],
        out_specs=bs,
        out_shape=jax.ShapeDtypeStruct((n, H), jnp.float32),
    )(agg, inv, h, wl, bl, wr)


def _sage2_body(a1, i1, a2, i2, h, wl1, bl1, wr1, wl2, bl2, wr2, o):
    m1 = a1[...] * i1[...]
    m2 = a2[...] * i2[...]
    hh = h[...]
    t = (jnp.dot(m1, wl1[...], preferred_element_type=jnp.float32) + bl1[...]
         + jnp.dot(hh, wr1[...], preferred_element_type=jnp.float32)
         + jnp.dot(m2, wl2[...], preferred_element_type=jnp.float32) + bl2[...]
         + jnp.dot(hh, wr2[...], preferred_element_type=jnp.float32))
    o[...] = jnp.maximum(t * 0.5, 0.0)


def _tc_sage2(a1, i1, a2, i2, h, wl1, bl1, wr1, wl2, bl2, wr2, n):
    bs = pl.BlockSpec((BM, H), lambda i: (i, 0))
    ws = pl.BlockSpec((H, H), lambda i: (0, 0))
    vs = pl.BlockSpec((1, H), lambda i: (0, 0))
    return pl.pallas_call(
        _sage2_body,
        grid=(n // BM,),
        in_specs=[bs, bs, bs, bs, bs, ws, vs, ws, ws, vs, ws],
        out_specs=bs,
        out_shape=jax.ShapeDtypeStruct((n, H), jnp.float32),
    )(a1, i1, a2, i2, h, wl1, bl1, wr1, wl2, bl2, wr2)


def _head_body(h, w1, b1, w2r, b2, o):
    x1 = jnp.maximum(jnp.dot(h[...], w1[...],
                             preferred_element_type=jnp.float32) + b1[...], 0.0)
    o[pl.ds(pl.program_id(0), 1), :] = (jnp.sum(x1 * w2r[...], axis=1)
                                        + b2[0, 0])[None, :]


def _tc_head(h, w1, b1, w2r, b2, n):
    return pl.pallas_call(
        _head_body,
        grid=(n // BM,),
        in_specs=[pl.BlockSpec((BM, H), lambda i: (i, 0)),
                  pl.BlockSpec((H, H // 2), lambda i: (0, 0)),
                  pl.BlockSpec((1, H // 2), lambda i: (0, 0)),
                  pl.BlockSpec((1, H // 2), lambda i: (0, 0)),
                  pl.BlockSpec((1, 1), lambda i: (0, 0))],
        out_specs=pl.BlockSpec((n // BM, BM), lambda i: (0, 0)),
        out_shape=jax.ShapeDtypeStruct((n // BM, BM), jnp.float32),
    )(h, w1, b1, w2r, b2).reshape(n)


EP_ORB = 102400
EP_SIB = 401408

_seg_star = _make_segsum(EP_ORB, NST, 1, counts=False)
_seg_host = _make_segsum(EP_ORB, NPL, 1, counts=False, zero_upper=True)
_seg_sib = _make_segsum(EP_SIB, NPL, 2, counts=False)
_cnt_star = _make_segsum(EP_ORB, NST, 1, counts=True)
_cnt_host = _make_segsum(EP_ORB, NPL, 1, counts=True, zero_upper=True)
_cnt_sib = _make_segsum(EP_SIB, NPL, 2, counts=True)


def kernel(x_planet, x_star, Wp, bp, Ws, bs, Wl, bl, Wr, W1, b1, W2, b2,
           ei_orbits, ei_hosts, ei_sibling):
    so, do = _pad_edges(ei_orbits, EP_ORB)
    sh, dh = _pad_edges(ei_hosts, EP_ORB)
    ss, ds_ = _pad_edges(ei_sibling, EP_SIB)

    hp = _tc_proj(x_planet, Wp, bp[None, :], NPL)
    hs = _tc_proj(x_star, Ws, bs[None, :], NST)

    inv_o = _cnt_star(hp, so, do)
    inv_h = _cnt_host(hs, sh, dh)
    inv_s = _cnt_sib(hp, ss, ds_)

    for l in range(NLAYERS):
        agg_s = _seg_star(hp, so, do)
        agg_p1 = _seg_host(hs, sh, dh)
        agg_p2 = _seg_sib(hp, ss, ds_)
        new_hs = _tc_sage1(agg_s, inv_o, hs,
                           Wl[l, 0], bl[l, 0][None, :], Wr[l, 0], NST)
        hp = _tc_sage2(agg_p1, inv_h, agg_p2, inv_s, hp,
                       Wl[l, 1], bl[l, 1][None, :], Wr[l, 1],
                       Wl[l, 2], bl[l, 2][None, :], Wr[l, 2], NPL)
        hs = new_hs

    return _tc_head(hp, W1, b1[None, :], W2[:, 0][None, :], b2[None, :], NPL)


# speedup vs baseline: 1.2169x; 1.2169x over previous
import jax
import jax.numpy as jnp
from jax import lax
from jax.experimental import pallas as pl
from jax.experimental.pallas import tpu as pltpu
from jax.experimental.pallas import tpu_sc as plsc

NPL = 100000
NST = 50000
DF = 128
H = 64
NLAYERS = 3

NC = 2
NS = 16
CHUNK = 256
IDXW = 128
R = 25000
R_PAD = 26624
WB = 125
_MESH = plsc.VectorSubcoreMesh(core_axis_name="c", subcore_axis_name="s")


def _fill(ref, nrows, val):
    def body(i, _):
        for j in range(H // 16):
            ref[i, pl.ds(j * 16, 16)] = jnp.full((16,), val, jnp.float32)
        return 0
    lax.fori_loop(0, nrows, body, 0)


def _make_segsum(e_pad, n_dst, nq, counts, zero_upper=False):
    chunks_per_tile = e_pad // (NS * CHUNK)
    wb_per_q = R // WB
    B = 128

    def body(h, srci, dsti, out, acc, srcv, dstv, csrc, cd2, srow, d2row,
             rows, sem):
        c = lax.axis_index("c")
        s = lax.axis_index("s")
        my_rows = R_PAD // NS
        iota = lax.iota(jnp.int32, 16)

        def stage(sl, off):
            for i in range(B // 16):
                d2row[sl, pl.ds(i * 16, 16)] = cd2[pl.ds(off + i * 16, 16)]
            if not counts:
                for i in range(B // 16):
                    srow[sl, pl.ds(i * 16, 16)] = csrc[pl.ds(off + i * 16, 16)]
                pltpu.async_copy(h.at[srow.at[sl]],
                                 rows.at[pl.ds(sl * B, B)], sem)

        def drain(sl):
            if not counts:
                pltpu.make_async_copy(h.at[pl.ds(0, B)],
                                      rows.at[pl.ds(sl * B, B)], sem).wait()
            pltpu.sync_copy(rows.at[pl.ds(sl * B, B)], acc.at[d2row.at[sl]],
                            add=True)

        def pipe_fire(t, off):
            @pl.when(t > 0)
            def _():
                drain((t - 1) & 1)
            stage(t & 1, off)

        for q in range(nq):
            base = (q * NC + c) * R
            _fill(rows, 128, 0.0)
            for z in range(my_rows // 128):
                pltpu.sync_copy(rows.at[pl.ds(0, 128)],
                                acc.at[pl.ds(s * my_rows + z * 128, 128)])
            if counts:
                _fill(rows, 2 * B, 1.0)
            plsc.subcore_barrier()

            def echunk(k, carry):
                noff, t = carry
                rb = (s * chunks_per_tile + k) * (CHUNK // IDXW)
                if not counts:
                    pltpu.sync_copy(srci.at[pl.ds(rb, CHUNK // IDXW)], srcv)
                pltpu.sync_copy(dsti.at[pl.ds(rb, CHUNK // IDXW)], dstv)
                for j in range(CHUNK // IDXW):
                    for i in range(IDXW // 16):
                        d = dstv[j, pl.ds(i * 16, 16)] - base
                        ok = (d >= 0) & (d < R)
                        lane = lax.iota(jnp.int32, 16)
                        _, dsrt, _ = plsc.sort_key_val(lane, d, mask=ok)
                        cd2[pl.ds(noff, 16)] = dsrt
                        if not counts:
                            _, ssrt, _ = plsc.sort_key_val(
                                lane, srcv[j, pl.ds(i * 16, 16)], mask=ok)
                            csrc[pl.ds(noff, 16)] = ssrt
                        noff = noff + jnp.sum(ok.astype(jnp.int32))
                @pl.when(noff >= B)
                def _():
                    pipe_fire(t, 0)
                @pl.when(noff >= 2 * B)
                def _():
                    pipe_fire(t + 1, B)
                nf = jnp.where(noff >= 2 * B, 2,
                               jnp.where(noff >= B, 1, 0)).astype(jnp.int32)
                drained = nf * B
                for i in range(B // 16):
                    v = cd2[pl.ds(drained + i * 16, 16)]
                    cd2[pl.ds(i * 16, 16)] = v
                    if not counts:
                        v2 = csrc[pl.ds(drained + i * 16, 16)]
                        csrc[pl.ds(i * 16, 16)] = v2
                return (noff - drained, t + nf)
            noff, t = lax.fori_loop(0, chunks_per_tile, echunk,
                                    (jnp.int32(0), jnp.int32(0)))
            @pl.when(t > 0)
            def _():
                drain((t - 1) & 1)
            for i in range(B // 16):
                iota = lax.iota(jnp.int32, 16)
                lane = i * 16 + iota
                keep = lane < noff
                junk = R + s * 84 + i * 16 + iota
                cd2[pl.ds(i * 16, 16)] = jnp.where(keep,
                                                   cd2[pl.ds(i * 16, 16)], junk)
                if not counts:
                    csrc[pl.ds(i * 16, 16)] = jnp.where(
                        keep, csrc[pl.ds(i * 16, 16)], 0)
            stage(t & 1, 0)
            drain(t & 1)
            plsc.subcore_barrier()

            for k in range((wb_per_q + NS - 1) // NS):
                cid = k * NS + s
                @pl.when(cid < wb_per_q)
                def _():
                    if counts:
                        pltpu.sync_copy(acc.at[pl.ds(cid * WB, WB)],
                                        rows.at[pl.ds(0, WB)])
                        def inv(i, _):
                            for j in range(H // 16):
                                x = rows[i, pl.ds(j * 16, 16)]
                                rows[i, pl.ds(j * 16, 16)] = 1.0 / jnp.maximum(x, 1.0)
                            return 0
                        lax.fori_loop(0, WB, inv, 0)
                        pltpu.sync_copy(rows.at[pl.ds(0, WB)],
                                        out.at[pl.ds(base + cid * WB, WB)])
                    else:
                        pltpu.sync_copy(acc.at[pl.ds(cid * WB, WB)],
                                        out.at[pl.ds(base + cid * WB, WB)])
            plsc.subcore_barrier()

        if zero_upper:
            zlen = (n_dst - nq * NC * R) // NC
            zbase = nq * NC * R + c * zlen
            _fill(rows, WB, 0.0)
            for k in range((zlen // WB + NS - 1) // NS):
                cid = k * NS + s
                @pl.when(cid < zlen // WB)
                def _():
                    pltpu.sync_copy(rows.at[pl.ds(0, WB)],
                                    out.at[pl.ds(zbase + cid * WB, WB)])

    return pl.kernel(
        body,
        out_type=jax.ShapeDtypeStruct((n_dst, H), jnp.float32),
        mesh=_MESH,
        compiler_params=pltpu.CompilerParams(use_tc_tiling_on_sc=False,
                                             needs_layout_passes=False),
        scratch_types=[
            pltpu.VMEM_SHARED((R_PAD, H), jnp.float32),
            pltpu.VMEM((CHUNK // IDXW, IDXW), jnp.int32),
            pltpu.VMEM((CHUNK // IDXW, IDXW), jnp.int32),
            pltpu.VMEM((528,), jnp.int32),
            pltpu.VMEM((528,), jnp.int32),
            pltpu.VMEM((2, 128), jnp.int32),
            pltpu.VMEM((2, 128), jnp.int32),
            pltpu.VMEM((256, H), jnp.float32),
            pltpu.SemaphoreType.DMA,
        ],
    )


def _pad_edges(ei, e_pad):
    src = ei[0]
    dst = ei[1]
    e = src.shape[0]
    pad = e_pad - e
    src = jnp.pad(src, (0, pad))
    dst = jnp.pad(dst, (0, pad), constant_values=jnp.int32(1 << 28))
    return src.reshape(e_pad // IDXW, IDXW), dst.reshape(e_pad // IDXW, IDXW)


BM = 2000


def _proj_body(x, w, b, o):
    o[...] = jnp.maximum(jnp.dot(x[...], w[...],
                                 preferred_element_type=jnp.float32) + b[...], 0.0)


def _tc_proj(x, w, b, n):
    return pl.pallas_call(
        _proj_body,
        grid=(n // BM,),
        in_specs=[pl.BlockSpec((BM, DF), lambda i: (i, 0)),
                  pl.BlockSpec((DF, H), lambda i: (0, 0)),
                  pl.BlockSpec((1, H), lambda i: (0, 0))],
        out_specs=pl.BlockSpec((BM, H), lambda i: (i, 0)),
        out_shape=jax.ShapeDtypeStruct((n, H), jnp.float32),
    )(x, w, b)


def _sage1_body(agg, inv, h, wl, bl, wr, o):
    m = agg[...] * inv[...]
    o[...] = jnp.maximum(
        jnp.dot(m, wl[...], preferred_element_type=jnp.float32) + bl[...]
        + jnp.dot(h[...], wr[...], preferred_element_type=jnp.float32), 0.0)


def _tc_sage1(agg, inv, h, wl, bl, wr, n):
    bs = pl.BlockSpec((BM, H), lambda i: (i, 0))
    ws = pl.BlockSpec((H, H), lambda i: (0, 0))
    return pl.pallas_call(
        _sage1_body,
        grid=(n // BM,),
        in_specs=[bs, bs, bs, ws, pl.BlockSpec((1, H), lambda i: (0, 0)), ws],
        out_specs=bs,
        out_shape=jax.ShapeDtypeStruct((n, H), jnp.float32),
    )(agg, inv, h, wl, bl, wr)


def _sage2_body(a1, i1, a2, i2, h, wl1, bl1, wr1, wl2, bl2, wr2, o):
    m1 = a1[...] * i1[...]
    m2 = a2[...] * i2[...]
    hh = h[...]
    t = (jnp.dot(m1, wl1[...], preferred_element_type=jnp.float32) + bl1[...]
         + jnp.dot(hh, wr1[...], preferred_element_type=jnp.float32)
         + jnp.dot(m2, wl2[...], preferred_element_type=jnp.float32) + bl2[...]
         + jnp.dot(hh, wr2[...], preferred_element_type=jnp.float32))
    o[...] = jnp.maximum(t * 0.5, 0.0)


def _tc_sage2(a1, i1, a2, i2, h, wl1, bl1, wr1, wl2, bl2, wr2, n):
    bs = pl.BlockSpec((BM, H), lambda i: (i, 0))
    ws = pl.BlockSpec((H, H), lambda i: (0, 0))
    vs = pl.BlockSpec((1, H), lambda i: (0, 0))
    return pl.pallas_call(
        _sage2_body,
        grid=(n // BM,),
        in_specs=[bs, bs, bs, bs, bs, ws, vs, ws, ws, vs, ws],
        out_specs=bs,
        out_shape=jax.ShapeDtypeStruct((n, H), jnp.float32),
    )(a1, i1, a2, i2, h, wl1, bl1, wr1, wl2, bl2, wr2)


def _head_body(h, w1, b1, w2r, b2, o):
    x1 = jnp.maximum(jnp.dot(h[...], w1[...],
                             preferred_element_type=jnp.float32) + b1[...], 0.0)
    o[pl.ds(pl.program_id(0), 1), :] = (jnp.sum(x1 * w2r[...], axis=1)
                                        + b2[0, 0])[None, :]


def _tc_head(h, w1, b1, w2r, b2, n):
    return pl.pallas_call(
        _head_body,
        grid=(n // BM,),
        in_specs=[pl.BlockSpec((BM, H), lambda i: (i, 0)),
                  pl.BlockSpec((H, H // 2), lambda i: (0, 0)),
                  pl.BlockSpec((1, H // 2), lambda i: (0, 0)),
                  pl.BlockSpec((1, H // 2), lambda i: (0, 0)),
                  pl.BlockSpec((1, 1), lambda i: (0, 0))],
        out_specs=pl.BlockSpec((n // BM, BM), lambda i: (0, 0)),
        out_shape=jax.ShapeDtypeStruct((n // BM, BM), jnp.float32),
    )(h, w1, b1, w2r, b2).reshape(n)


EP_ORB = 102400
EP_SIB = 401408

_seg_star = _make_segsum(EP_ORB, NST, 1, counts=False)
_seg_host = _make_segsum(EP_ORB, NPL, 1, counts=False, zero_upper=True)
_seg_sib = _make_segsum(EP_SIB, NPL, 2, counts=False)
_cnt_star = _make_segsum(EP_ORB, NST, 1, counts=True)
_cnt_host = _make_segsum(EP_ORB, NPL, 1, counts=True, zero_upper=True)
_cnt_sib = _make_segsum(EP_SIB, NPL, 2, counts=True)


def kernel(x_planet, x_star, Wp, bp, Ws, bs, Wl, bl, Wr, W1, b1, W2, b2,
           ei_orbits, ei_hosts, ei_sibling):
    so, do = _pad_edges(ei_orbits, EP_ORB)
    sh, dh = _pad_edges(ei_hosts, EP_ORB)
    ss, ds_ = _pad_edges(ei_sibling, EP_SIB)

    hp = _tc_proj(x_planet, Wp, bp[None, :], NPL)
    hs = _tc_proj(x_star, Ws, bs[None, :], NST)

    inv_o = _cnt_star(hp, so, do)
    inv_h = _cnt_host(hs, sh, dh)
    inv_s = _cnt_sib(hp, ss, ds_)

    for l in range(NLAYERS):
        agg_s = _seg_star(hp, so, do)
        agg_p1 = _seg_host(hs, sh, dh)
        agg_p2 = _seg_sib(hp, ss, ds_)
        new_hs = _tc_sage1(agg_s, inv_o, hs,
                           Wl[l, 0], bl[l, 0][None, :], Wr[l, 0], NST)
        hp = _tc_sage2(agg_p1, inv_h, agg_p2, inv_s, hp,
                       Wl[l, 1], bl[l, 1][None, :], Wr[l, 1],
                       Wl[l, 2], bl[l, 2][None, :], Wr[l, 2], NPL)
        hs = new_hs

    return _tc_head(hp, W1, b1[None, :], W2[:, 0][None, :], b2[None, :], NPL)


# speedup vs baseline: 1.2746x; 1.0474x over previous
import jax
import jax.numpy as jnp
from jax import lax
from jax.experimental import pallas as pl
from jax.experimental.pallas import tpu as pltpu
from jax.experimental.pallas import tpu_sc as plsc

NPL = 100000
NST = 50000
DF = 128
H = 64
NLAYERS = 3

NC = 2
NS = 16
CHUNK = 256
IDXW = 128
R = 25000
R_PAD = 26624
WB = 125
_MESH = plsc.VectorSubcoreMesh(core_axis_name="c", subcore_axis_name="s")


def _fill(ref, nrows, val):
    def body(i, _):
        for j in range(H // 16):
            ref[i, pl.ds(j * 16, 16)] = jnp.full((16,), val, jnp.float32)
        return 0
    lax.fori_loop(0, nrows, body, 0)


def _make_segsum(e_pad, n_dst, nq, counts, zero_upper=False):
    chunks_per_tile = e_pad // (NS * CHUNK)
    wb_per_q = R // WB
    B = 128

    def body(h, srci, dsti, out, acc, srcv, dstv, csrc, cd2, srow, d2row,
             rows, sem):
        c = lax.axis_index("c")
        s = lax.axis_index("s")
        my_rows = R_PAD // NS
        iota = lax.iota(jnp.int32, 16)

        def stage(sl, off):
            for i in range(B // 16):
                v = cd2[pl.ds(off + i * 16, 16)]
                if counts:
                    d2row[sl, pl.ds(i * 16, 16)] = v
                else:
                    d2row[sl, pl.ds(i * 16, 16)] = lax.shift_right_logical(v, 17)
                    srow[sl, pl.ds(i * 16, 16)] = v & 0x1FFFF
            if not counts:
                pltpu.async_copy(h.at[srow.at[sl]],
                                 rows.at[pl.ds(sl * B, B)], sem)

        def drain(sl):
            if not counts:
                pltpu.make_async_copy(h.at[pl.ds(0, B)],
                                      rows.at[pl.ds(sl * B, B)], sem).wait()
            pltpu.sync_copy(rows.at[pl.ds(sl * B, B)], acc.at[d2row.at[sl]],
                            add=True)

        def pipe_fire(t, off):
            @pl.when(t > 0)
            def _():
                drain((t - 1) & 1)
            stage(t & 1, off)

        for q in range(nq):
            base = (q * NC + c) * R
            _fill(rows, 128, 0.0)
            for z in range(my_rows // 128):
                pltpu.sync_copy(rows.at[pl.ds(0, 128)],
                                acc.at[pl.ds(s * my_rows + z * 128, 128)])
            if counts:
                _fill(rows, 2 * B, 1.0)
            plsc.subcore_barrier()

            def echunk(k, carry):
                noff, t = carry
                rb = (s * chunks_per_tile + k) * (CHUNK // IDXW)
                if not counts:
                    pltpu.sync_copy(srci.at[pl.ds(rb, CHUNK // IDXW)], srcv)
                pltpu.sync_copy(dsti.at[pl.ds(rb, CHUNK // IDXW)], dstv)
                for j in range(CHUNK // IDXW):
                    for i in range(IDXW // 16):
                        d = dstv[j, pl.ds(i * 16, 16)] - base
                        ok = (d >= 0) & (d < R)
                        lane = lax.iota(jnp.int32, 16)
                        if counts:
                            pv = d
                        else:
                            pv = (lax.shift_left(d, 17)
                                  | srcv[j, pl.ds(i * 16, 16)])
                        _, psrt, _ = plsc.sort_key_val(lane, pv, mask=ok)
                        cd2[pl.ds(noff, 16)] = psrt
                        noff = noff + jnp.sum(ok.astype(jnp.int32))
                @pl.when(noff >= B)
                def _():
                    pipe_fire(t, 0)
                @pl.when(noff >= 2 * B)
                def _():
                    pipe_fire(t + 1, B)
                nf = jnp.where(noff >= 2 * B, 2,
                               jnp.where(noff >= B, 1, 0)).astype(jnp.int32)
                drained = nf * B
                for i in range(B // 16):
                    v = cd2[pl.ds(drained + i * 16, 16)]
                    cd2[pl.ds(i * 16, 16)] = v
                return (noff - drained, t + nf)
            noff, t = lax.fori_loop(0, chunks_per_tile, echunk,
                                    (jnp.int32(0), jnp.int32(0)))
            @pl.when(t > 0)
            def _():
                drain((t - 1) & 1)
            for i in range(B // 16):
                iota = lax.iota(jnp.int32, 16)
                lane = i * 16 + iota
                keep = lane < noff
                junk = R + s * 84 + i * 16 + iota
                if not counts:
                    junk = lax.shift_left(junk, 17)
                cd2[pl.ds(i * 16, 16)] = jnp.where(keep,
                                                   cd2[pl.ds(i * 16, 16)], junk)
            stage(t & 1, 0)
            drain(t & 1)
            plsc.subcore_barrier()

            for k in range((wb_per_q + NS - 1) // NS):
                cid = k * NS + s
                @pl.when(cid < wb_per_q)
                def _():
                    if counts:
                        pltpu.sync_copy(acc.at[pl.ds(cid * WB, WB)],
                                        rows.at[pl.ds(0, WB)])
                        def inv(i, _):
                            for j in range(H // 16):
                                x = rows[i, pl.ds(j * 16, 16)]
                                rows[i, pl.ds(j * 16, 16)] = 1.0 / jnp.maximum(x, 1.0)
                            return 0
                        lax.fori_loop(0, WB, inv, 0)
                        pltpu.sync_copy(rows.at[pl.ds(0, WB)],
                                        out.at[pl.ds(base + cid * WB, WB)])
                    else:
                        pltpu.sync_copy(acc.at[pl.ds(cid * WB, WB)],
                                        out.at[pl.ds(base + cid * WB, WB)])
            plsc.subcore_barrier()

        if zero_upper:
            zlen = (n_dst - nq * NC * R) // NC
            zbase = nq * NC * R + c * zlen
            _fill(rows, WB, 0.0)
            for k in range((zlen // WB + NS - 1) // NS):
                cid = k * NS + s
                @pl.when(cid < zlen // WB)
                def _():
                    pltpu.sync_copy(rows.at[pl.ds(0, WB)],
                                    out.at[pl.ds(zbase + cid * WB, WB)])

    return pl.kernel(
        body,
        out_type=jax.ShapeDtypeStruct((n_dst, H), jnp.float32),
        mesh=_MESH,
        compiler_params=pltpu.CompilerParams(use_tc_tiling_on_sc=False,
                                             needs_layout_passes=False),
        scratch_types=[
            pltpu.VMEM_SHARED((R_PAD, H), jnp.float32),
            pltpu.VMEM((CHUNK // IDXW, IDXW), jnp.int32),
            pltpu.VMEM((CHUNK // IDXW, IDXW), jnp.int32),
            pltpu.VMEM((528,), jnp.int32),
            pltpu.VMEM((528,), jnp.int32),
            pltpu.VMEM((2, 128), jnp.int32),
            pltpu.VMEM((2, 128), jnp.int32),
            pltpu.VMEM((256, H), jnp.float32),
            pltpu.SemaphoreType.DMA,
        ],
    )


def _pad_edges(ei, e_pad):
    src = ei[0]
    dst = ei[1]
    e = src.shape[0]
    pad = e_pad - e
    src = jnp.pad(src, (0, pad))
    dst = jnp.pad(dst, (0, pad), constant_values=jnp.int32(1 << 28))
    return src.reshape(e_pad // IDXW, IDXW), dst.reshape(e_pad // IDXW, IDXW)


BM = 2000


def _proj_body(x, w, b, o):
    o[...] = jnp.maximum(jnp.dot(x[...], w[...],
                                 preferred_element_type=jnp.float32) + b[...], 0.0)


def _tc_proj(x, w, b, n):
    return pl.pallas_call(
        _proj_body,
        grid=(n // BM,),
        in_specs=[pl.BlockSpec((BM, DF), lambda i: (i, 0)),
                  pl.BlockSpec((DF, H), lambda i: (0, 0)),
                  pl.BlockSpec((1, H), lambda i: (0, 0))],
        out_specs=pl.BlockSpec((BM, H), lambda i: (i, 0)),
        out_shape=jax.ShapeDtypeStruct((n, H), jnp.float32),
    )(x, w, b)


def _sage1_body(agg, inv, h, wl, bl, wr, o):
    m = agg[...] * inv[...]
    o[...] = jnp.maximum(
        jnp.dot(m, wl[...], preferred_element_type=jnp.float32) + bl[...]
        + jnp.dot(h[...], wr[...], preferred_element_type=jnp.float32), 0.0)


def _tc_sage1(agg, inv, h, wl, bl, wr, n):
    bs = pl.BlockSpec((BM, H), lambda i: (i, 0))
    ws = pl.BlockSpec((H, H), lambda i: (0, 0))
    return pl.pallas_call(
        _sage1_body,
        grid=(n // BM,),
        in_specs=[bs, bs, bs, ws, pl.BlockSpec((1, H), lambda i: (0, 0)), ws],
        out_specs=bs,
        out_shape=jax.ShapeDtypeStruct((n, H), jnp.float32),
    )(agg, inv, h, wl, bl, wr)


def _sage2_body(a1, i1, a2, i2, h, wl1, bl1, wr1, wl2, bl2, wr2, o):
    m1 = a1[...] * i1[...]
    m2 = a2[...] * i2[...]
    hh = h[...]
    t = (jnp.dot(m1, wl1[...], preferred_element_type=jnp.float32) + bl1[...]
         + jnp.dot(hh, wr1[...], preferred_element_type=jnp.float32)
         + jnp.dot(m2, wl2[...], preferred_element_type=jnp.float32) + bl2[...]
         + jnp.dot(hh, wr2[...], preferred_element_type=jnp.float32))
    o[...] = jnp.maximum(t * 0.5, 0.0)


def _tc_sage2(a1, i1, a2, i2, h, wl1, bl1, wr1, wl2, bl2, wr2, n):
    bs = pl.BlockSpec((BM, H), lambda i: (i, 0))
    ws = pl.BlockSpec((H, H), lambda i: (0, 0))
    vs = pl.BlockSpec((1, H), lambda i: (0, 0))
    return pl.pallas_call(
        _sage2_body,
        grid=(n // BM,),
        in_specs=[bs, bs, bs, bs, bs, ws, vs, ws, ws, vs, ws],
        out_specs=bs,
        out_shape=jax.ShapeDtypeStruct((n, H), jnp.float32),
    )(a1, i1, a2, i2, h, wl1, bl1, wr1, wl2, bl2, wr2)


def _head_body(h, w1, b1, w2r, b2, o):
    x1 = jnp.maximum(jnp.dot(h[...], w1[...],
                             preferred_element_type=jnp.float32) + b1[...], 0.0)
    o[pl.ds(pl.program_id(0), 1), :] = (jnp.sum(x1 * w2r[...], axis=1)
                                        + b2[0, 0])[None, :]


def _tc_head(h, w1, b1, w2r, b2, n):
    return pl.pallas_call(
        _head_body,
        grid=(n // BM,),
        in_specs=[pl.BlockSpec((BM, H), lambda i: (i, 0)),
                  pl.BlockSpec((H, H // 2), lambda i: (0, 0)),
                  pl.BlockSpec((1, H // 2), lambda i: (0, 0)),
                  pl.BlockSpec((1, H // 2), lambda i: (0, 0)),
                  pl.BlockSpec((1, 1), lambda i: (0, 0))],
        out_specs=pl.BlockSpec((n // BM, BM), lambda i: (0, 0)),
        out_shape=jax.ShapeDtypeStruct((n // BM, BM), jnp.float32),
    )(h, w1, b1, w2r, b2).reshape(n)


EP_ORB = 102400
EP_SIB = 401408

_seg_star = _make_segsum(EP_ORB, NST, 1, counts=False)
_seg_host = _make_segsum(EP_ORB, NPL, 1, counts=False, zero_upper=True)
_seg_sib = _make_segsum(EP_SIB, NPL, 2, counts=False)
_cnt_star = _make_segsum(EP_ORB, NST, 1, counts=True)
_cnt_host = _make_segsum(EP_ORB, NPL, 1, counts=True, zero_upper=True)
_cnt_sib = _make_segsum(EP_SIB, NPL, 2, counts=True)


def kernel(x_planet, x_star, Wp, bp, Ws, bs, Wl, bl, Wr, W1, b1, W2, b2,
           ei_orbits, ei_hosts, ei_sibling):
    so, do = _pad_edges(ei_orbits, EP_ORB)
    sh, dh = _pad_edges(ei_hosts, EP_ORB)
    ss, ds_ = _pad_edges(ei_sibling, EP_SIB)

    hp = _tc_proj(x_planet, Wp, bp[None, :], NPL)
    hs = _tc_proj(x_star, Ws, bs[None, :], NST)

    inv_o = _cnt_star(hp, so, do)
    inv_h = _cnt_host(hs, sh, dh)
    inv_s = _cnt_sib(hp, ss, ds_)

    for l in range(NLAYERS):
        agg_s = _seg_star(hp, so, do)
        agg_p1 = _seg_host(hs, sh, dh)
        agg_p2 = _seg_sib(hp, ss, ds_)
        new_hs = _tc_sage1(agg_s, inv_o, hs,
                           Wl[l, 0], bl[l, 0][None, :], Wr[l, 0], NST)
        hp = _tc_sage2(agg_p1, inv_h, agg_p2, inv_s, hp,
                       Wl[l, 1], bl[l, 1][None, :], Wr[l, 1],
                       Wl[l, 2], bl[l, 2][None, :], Wr[l, 2], NPL)
        hs = new_hs

    return _tc_head(hp, W1, b1[None, :], W2[:, 0][None, :], b2[None, :], NPL)
